# Initial kernel scaffold; baseline (speedup 1.0000x reference)
import functools

import jax
import jax.numpy as jnp
from jax import lax
from jax.experimental import pallas as pl
from jax.experimental.pallas import tpu as pltpu
from jax.experimental.pallas import tpu_sc as plsc

B, H, C = 64, 720, 512
EPS = 1e-4
LANES = 16
NUM_CORES = 2
NUM_SUBCORES = 16
NW = NUM_CORES * NUM_SUBCORES
CG = C // LANES
TASKS = B * CG
TASKS_PER_W = TASKS // NW


def _tau_body(fst_ref, w_ref, b_ref, tri_ref, t_ref):
    x = fst_ref[0]
    raw = lax.dot_general(x, w_ref[...], (((1,), (1,)), ((), ())),
                          preferred_element_type=jnp.float32) + b_ref[...]
    sp = jnp.maximum(raw, 0.0) + jnp.log1p(jnp.exp(-jnp.abs(raw)))
    v = sp + EPS
    tau = lax.dot_general(tri_ref[...], v, (((1,), (0,)), ((), ())),
                          preferred_element_type=jnp.float32)
    tau_min = tau[0:1, :]
    tau_max = tau[H - 1:H, :]
    t = (tau - tau_min) / (tau_max - tau_min + 1e-8) * (H - 1)
    t_ref[0] = jnp.clip(t, 0.0, float(H - 1))


def _compute_t(fst, w, b2, tri):
    return pl.pallas_call(
        _tau_body,
        grid=(B,),
        in_specs=[
            pl.BlockSpec((1, H, C), lambda i: (i, 0, 0)),
            pl.BlockSpec((C, C), lambda i: (0, 0)),
            pl.BlockSpec((1, C), lambda i: (0, 0)),
            pl.BlockSpec((H, H), lambda i: (0, 0)),
        ],
        out_specs=pl.BlockSpec((1, H, C), lambda i: (i, 0, 0)),
        out_shape=jax.ShapeDtypeStruct((B, H, C), jnp.float32),
    )(fst, w, b2, tri)


def _gather_body(yb_hbm, t_hbm, out_hbm, ytab, ttab, otab):
    cid = lax.axis_index("c")
    sid = lax.axis_index("s")
    wid = sid * NUM_CORES + cid
    lane = lax.iota(jnp.int32, LANES)

    def task_body(k, carry):
        task = wid * TASKS_PER_W + k
        bi = task // CG
        c0 = (task % CG) * LANES
        pltpu.sync_copy(yb_hbm.at[bi, :, pl.ds(c0, LANES)], ytab)
        pltpu.sync_copy(t_hbm.at[bi, :, pl.ds(c0, LANES)], ttab)

        def h_body(h, c):
            t = ttab[h]
            fi = jnp.minimum(t.astype(jnp.int32), H - 2)
            yf = plsc.load_gather(ytab, [fi, lane])
            yc = plsc.load_gather(ytab, [fi + 1, lane])
            frac = t - fi.astype(jnp.float32)
            otab[h] = yf + (yc - yf) * frac
            return c

        lax.fori_loop(0, H, h_body, 0)
        pltpu.sync_copy(otab, out_hbm.at[bi, :, pl.ds(c0, LANES)])
        return carry

    lax.fori_loop(0, TASKS_PER_W, task_body, 0)


def _warp_gather(y_base, t):
    mesh = plsc.VectorSubcoreMesh(core_axis_name="c", subcore_axis_name="s")
    return pl.kernel(
        _gather_body,
        out_type=jax.ShapeDtypeStruct((B, H, C), jnp.float32),
        mesh=mesh,
        scratch_types=[
            pltpu.VMEM((H, LANES), jnp.float32),
            pltpu.VMEM((H, LANES), jnp.float32),
            pltpu.VMEM((H, LANES), jnp.float32),
        ],
    )(y_base, t)


def kernel(y_norm, y_base, future_state_hat, future_state_time,
           mu_base_fut, std_base_fut, base_time_mean, base_time_std, W, b):
    tri = jnp.tri(H, dtype=jnp.float32)
    t = _compute_t(future_state_time, W, b.reshape(1, C), tri)
    return _warp_gather(y_base, t)


# speedup vs baseline: 2.9285x; 2.9285x over previous
import functools

import jax
import jax.numpy as jnp
from jax import lax
from jax.experimental import pallas as pl
from jax.experimental.pallas import tpu as pltpu
from jax.experimental.pallas import tpu_sc as plsc

B, H, C = 64, 720, 512
EPS = 1e-4
LANES = 16
NUM_CORES = 2
NUM_SUBCORES = 16
NW = NUM_CORES * NUM_SUBCORES
CG = C // LANES
TASKS = B * CG
TASKS_PER_W = TASKS // NW


def _tau_body(fst_ref, w_ref, b_ref, t_ref):
    x = fst_ref[0]
    raw = lax.dot_general(x, w_ref[...], (((1,), (1,)), ((), ())),
                          preferred_element_type=jnp.float32) + b_ref[...]
    sp = jnp.maximum(raw, 0.0) + jnp.log1p(jnp.exp(-jnp.abs(raw)))
    v = sp + EPS
    tau = v
    k = 1
    while k < H:
        shifted = jnp.concatenate(
            [jnp.zeros((k, C), jnp.float32), tau[:H - k]], axis=0)
        tau = tau + shifted
        k *= 2
    tau_min = tau[0:1, :]
    tau_max = tau[H - 1:H, :]
    t = (tau - tau_min) / (tau_max - tau_min + 1e-8) * (H - 1)
    t_ref[0] = jnp.clip(t, 0.0, float(H - 1))


def _compute_t(fst, w, b2):
    return pl.pallas_call(
        _tau_body,
        grid=(B,),
        in_specs=[
            pl.BlockSpec((1, H, C), lambda i: (i, 0, 0)),
            pl.BlockSpec((C, C), lambda i: (0, 0)),
            pl.BlockSpec((1, C), lambda i: (0, 0)),
        ],
        out_specs=pl.BlockSpec((1, H, C), lambda i: (i, 0, 0)),
        out_shape=jax.ShapeDtypeStruct((B, H, C), jnp.float32),
    )(fst, w, b2)


def _gather_body(yb_hbm, t_hbm, out_hbm, ytab, ttab, otab):
    cid = lax.axis_index("c")
    sid = lax.axis_index("s")
    wid = sid * NUM_CORES + cid
    lane = lax.iota(jnp.int32, LANES)

    def task_body(k, carry):
        task = wid * TASKS_PER_W + k
        bi = task // CG
        c0 = (task % CG) * LANES
        pltpu.sync_copy(yb_hbm.at[bi, :, pl.ds(c0, LANES)], ytab)
        pltpu.sync_copy(t_hbm.at[bi, :, pl.ds(c0, LANES)], ttab)

        def h_body(h, c):
            t = ttab[h]
            fi = jnp.minimum(t.astype(jnp.int32), H - 2)
            yf = plsc.load_gather(ytab, [fi, lane])
            yc = plsc.load_gather(ytab, [fi + 1, lane])
            frac = t - fi.astype(jnp.float32)
            otab[h] = yf + (yc - yf) * frac
            return c

        lax.fori_loop(0, H, h_body, 0)
        pltpu.sync_copy(otab, out_hbm.at[bi, :, pl.ds(c0, LANES)])
        return carry

    lax.fori_loop(0, TASKS_PER_W, task_body, 0)


def _warp_gather(y_base, t):
    mesh = plsc.VectorSubcoreMesh(core_axis_name="c", subcore_axis_name="s")
    return pl.kernel(
        _gather_body,
        out_type=jax.ShapeDtypeStruct((B, H, C), jnp.float32),
        mesh=mesh,
        scratch_types=[
            pltpu.VMEM((H, LANES), jnp.float32),
            pltpu.VMEM((H, LANES), jnp.float32),
            pltpu.VMEM((H, LANES), jnp.float32),
        ],
        compiler_params=pltpu.CompilerParams(use_tc_tiling_on_sc=False,
                                             needs_layout_passes=False),
    )(y_base, t)


def kernel(y_norm, y_base, future_state_hat, future_state_time,
           mu_base_fut, std_base_fut, base_time_mean, base_time_std, W, b):
    t = _compute_t(future_state_time, W, b.reshape(1, C))
    return _warp_gather(y_base, t)


# speedup vs baseline: 3.8807x; 1.3252x over previous
import functools

import jax
import jax.numpy as jnp
from jax import lax
from jax.experimental import pallas as pl
from jax.experimental.pallas import tpu as pltpu
from jax.experimental.pallas import tpu_sc as plsc

B, H, C = 64, 720, 512
EPS = 1e-4
LANES = 16
NUM_CORES = 2
NUM_SUBCORES = 16
NW = NUM_CORES * NUM_SUBCORES
CG = C // LANES
TASKS = B * CG
TASKS_PER_W = TASKS // NW


def _tau_body(fst_ref, w_ref, b_ref, t_ref):
    x = fst_ref[0]
    raw = lax.dot_general(x, w_ref[...], (((1,), (1,)), ((), ())),
                          preferred_element_type=jnp.float32) + b_ref[...]
    sp = jnp.maximum(raw, 0.0) + jnp.log1p(jnp.exp(-jnp.abs(raw)))
    v = sp + EPS
    tau = v
    k = 1
    while k < H:
        shifted = jnp.concatenate(
            [jnp.zeros((k, C), jnp.float32), tau[:H - k]], axis=0)
        tau = tau + shifted
        k *= 2
    tau_min = tau[0:1, :]
    tau_max = tau[H - 1:H, :]
    t = (tau - tau_min) / (tau_max - tau_min + 1e-8) * (H - 1)
    t_ref[0] = jnp.clip(t, 0.0, float(H - 1))


def _compute_t(fst, w, b2):
    return pl.pallas_call(
        _tau_body,
        grid=(B,),
        in_specs=[
            pl.BlockSpec((1, H, C), lambda i: (i, 0, 0)),
            pl.BlockSpec((C, C), lambda i: (0, 0)),
            pl.BlockSpec((1, C), lambda i: (0, 0)),
        ],
        out_specs=pl.BlockSpec((1, H, C), lambda i: (i, 0, 0)),
        out_shape=jax.ShapeDtypeStruct((B, H, C), jnp.float32),
    )(fst, w, b2)


def _gather_body(yb_hbm, t_hbm, out_hbm, ytab, ttab, otab):
    cid = lax.axis_index("c")
    sid = lax.axis_index("s")
    wid = sid * NUM_CORES + cid
    lane = lax.iota(jnp.int32, LANES)

    def task_body(k, carry):
        task = wid * TASKS_PER_W + k
        bi = task // CG
        c0 = (task % CG) * LANES
        pltpu.sync_copy(yb_hbm.at[bi, :, pl.ds(c0, LANES)], ytab)
        pltpu.sync_copy(t_hbm.at[bi, :, pl.ds(c0, LANES)], ttab)

        @plsc.parallel_loop(0, H, unroll=8)
        def h_body(h):
            t = ttab[h]
            fi = jnp.minimum(t.astype(jnp.int32), H - 2)
            yf = plsc.load_gather(ytab, [fi, lane])
            yc = plsc.load_gather(ytab, [fi + 1, lane])
            frac = t - fi.astype(jnp.float32)
            otab[h] = yf + (yc - yf) * frac
        pltpu.sync_copy(otab, out_hbm.at[bi, :, pl.ds(c0, LANES)])
        return carry

    lax.fori_loop(0, TASKS_PER_W, task_body, 0)


def _warp_gather(y_base, t):
    mesh = plsc.VectorSubcoreMesh(core_axis_name="c", subcore_axis_name="s")
    return pl.kernel(
        _gather_body,
        out_type=jax.ShapeDtypeStruct((B, H, C), jnp.float32),
        mesh=mesh,
        scratch_types=[
            pltpu.VMEM((H, LANES), jnp.float32),
            pltpu.VMEM((H, LANES), jnp.float32),
            pltpu.VMEM((H, LANES), jnp.float32),
        ],
        compiler_params=pltpu.CompilerParams(use_tc_tiling_on_sc=False,
                                             needs_layout_passes=False),
    )(y_base, t)


def kernel(y_norm, y_base, future_state_hat, future_state_time,
           mu_base_fut, std_base_fut, base_time_mean, base_time_std, W, b):
    t = _compute_t(future_state_time, W, b.reshape(1, C))
    return _warp_gather(y_base, t)


# speedup vs baseline: 5.3810x; 1.3866x over previous
import functools

import jax
import jax.numpy as jnp
from jax import lax
from jax.experimental import pallas as pl
from jax.experimental.pallas import tpu as pltpu
from jax.experimental.pallas import tpu_sc as plsc

B, H, C = 64, 720, 512
EPS = 1e-4
LANES = 16
NUM_CORES = 2
NUM_SUBCORES = 16
NW = NUM_CORES * NUM_SUBCORES
CG = C // LANES
TASKS = B * CG
TASKS_PER_W = TASKS // NW


def _tau_body(fst_ref, w_ref, b_ref, t_ref):
    x = fst_ref[0]
    raw = lax.dot_general(x, w_ref[...], (((1,), (1,)), ((), ())),
                          preferred_element_type=jnp.float32) + b_ref[...]
    sp = jnp.maximum(raw, 0.0) + jnp.log1p(jnp.exp(-jnp.abs(raw)))
    v = sp + EPS
    tau = v
    k = 1
    while k < H:
        shifted = jnp.concatenate(
            [jnp.zeros((k, C), jnp.float32), tau[:H - k]], axis=0)
        tau = tau + shifted
        k *= 2
    tau_min = tau[0:1, :]
    tau_max = tau[H - 1:H, :]
    t = (tau - tau_min) / (tau_max - tau_min + 1e-8) * (H - 1)
    t_ref[0] = jnp.clip(t, 0.0, float(H - 1))


def _compute_t(fst, w, b2):
    return pl.pallas_call(
        _tau_body,
        grid=(B,),
        in_specs=[
            pl.BlockSpec((1, H, C), lambda i: (i, 0, 0)),
            pl.BlockSpec((C, C), lambda i: (0, 0)),
            pl.BlockSpec((1, C), lambda i: (0, 0)),
        ],
        out_specs=pl.BlockSpec((1, H, C), lambda i: (i, 0, 0)),
        out_shape=jax.ShapeDtypeStruct((B, H, C), jnp.float32),
    )(fst, w, b2)


def _gather_body(yb_hbm, t_hbm, out_hbm, ybuf, tbuf, obuf, ysem, tsem, osem):
    cid = lax.axis_index("c")
    sid = lax.axis_index("s")
    wid = sid * NUM_CORES + cid
    base = wid * TASKS_PER_W
    lane = lax.iota(jnp.int32, LANES)

    def locate(k):
        task = base + k
        return task // CG, (task % CG) * LANES

    def in_descs(k, j):
        bi, c0 = locate(k)
        return (pltpu.make_async_copy(yb_hbm.at[bi, :, pl.ds(c0, LANES)],
                                      ybuf.at[j], ysem.at[j]),
                pltpu.make_async_copy(t_hbm.at[bi, :, pl.ds(c0, LANES)],
                                      tbuf.at[j], tsem.at[j]))

    def out_desc(k, j):
        bi, c0 = locate(k)
        return pltpu.make_async_copy(obuf.at[j],
                                     out_hbm.at[bi, :, pl.ds(c0, LANES)],
                                     osem.at[j])

    def start_in(k, j):
        for d in in_descs(k, j):
            d.start()

    y0, t0 = in_descs(0, 0)
    y0.start()
    t0.start()

    def pair_body(p, carry):
        for j in (0, 1):
            k = 2 * p + j

            @pl.when(k + 1 < TASKS_PER_W)
            def _():
                start_in(k + 1, 1 - j)

            yd, td = in_descs(k, j)
            yd.wait()
            td.wait()

            @pl.when(k >= 2)
            def _():
                out_desc(k - 2, j).wait()

            @plsc.parallel_loop(0, H, unroll=8)
            def h_body(h):
                t = tbuf[j, h]
                fi = jnp.minimum(t.astype(jnp.int32), H - 2)
                yf = plsc.load_gather(ybuf.at[j], [fi, lane])
                yc = plsc.load_gather(ybuf.at[j], [fi + 1, lane])
                frac = t - fi.astype(jnp.float32)
                obuf[j, h] = yf + (yc - yf) * frac

            out_desc(k, j).start()
        return carry

    lax.fori_loop(0, TASKS_PER_W // 2, pair_body, 0)
    out_desc(TASKS_PER_W - 2, 0).wait()
    out_desc(TASKS_PER_W - 1, 1).wait()


def _warp_gather(y_base, t):
    mesh = plsc.VectorSubcoreMesh(core_axis_name="c", subcore_axis_name="s")
    return pl.kernel(
        _gather_body,
        out_type=jax.ShapeDtypeStruct((B, H, C), jnp.float32),
        mesh=mesh,
        scratch_types=[
            pltpu.VMEM((2, H, LANES), jnp.float32),
            pltpu.VMEM((2, H, LANES), jnp.float32),
            pltpu.VMEM((2, H, LANES), jnp.float32),
            pltpu.SemaphoreType.DMA((2,)),
            pltpu.SemaphoreType.DMA((2,)),
            pltpu.SemaphoreType.DMA((2,)),
        ],
        compiler_params=pltpu.CompilerParams(use_tc_tiling_on_sc=False,
                                             needs_layout_passes=False),
    )(y_base, t)


def kernel(y_norm, y_base, future_state_hat, future_state_time,
           mu_base_fut, std_base_fut, base_time_mean, base_time_std, W, b):
    t = _compute_t(future_state_time, W, b.reshape(1, C))
    return _warp_gather(y_base, t)


# speedup vs baseline: 8.3557x; 1.5528x over previous
import functools

import jax
import jax.numpy as jnp
from jax import lax
from jax.experimental import pallas as pl
from jax.experimental.pallas import tpu as pltpu
from jax.experimental.pallas import tpu_sc as plsc

B, H, C = 64, 720, 512
EPS = 1e-4
LANES = 16
NUM_CORES = 2
NUM_SUBCORES = 16
NW = NUM_CORES * NUM_SUBCORES
CG = C // LANES
TASKS = B * CG
TASKS_PER_W = TASKS // NW


def _tau_body(fst_ref, w_ref, b_ref, t_ref):
    x = fst_ref[0]
    raw = lax.dot_general(x, w_ref[...], (((1,), (1,)), ((), ())),
                          preferred_element_type=jnp.float32) + b_ref[...]
    sp = jnp.maximum(raw, 0.0) + jnp.log1p(jnp.exp(-jnp.abs(raw)))
    v = sp + EPS
    tau = v
    k = 1
    while k < H:
        shifted = jnp.concatenate(
            [jnp.zeros((k, C), jnp.float32), tau[:H - k]], axis=0)
        tau = tau + shifted
        k *= 2
    tau_min = tau[0:1, :]
    tau_max = tau[H - 1:H, :]
    t = (tau - tau_min) / (tau_max - tau_min + 1e-8) * (H - 1)
    t_ref[0] = jnp.clip(t, 0.0, float(H - 1))


def _compute_t(fst, w, b2):
    return pl.pallas_call(
        _tau_body,
        grid=(B,),
        in_specs=[
            pl.BlockSpec((1, H, C), lambda i: (i, 0, 0)),
            pl.BlockSpec((C, C), lambda i: (0, 0)),
            pl.BlockSpec((1, C), lambda i: (0, 0)),
        ],
        out_specs=pl.BlockSpec((1, H, C), lambda i: (i, 0, 0)),
        out_shape=jax.ShapeDtypeStruct((B, H, C), jnp.float32),
    )(fst, w, b2)


SLAB = 128
NSLAB = C // SLAB
SC_TASKS = B * NSLAB
SC_TASKS_PER_W = SC_TASKS // NW
SB = 40
NSTRIP = H // SB
NGRP = SLAB // LANES


def _gather_body(yb_hbm, t_hbm, out_hbm, ytab, tbuf, obuf, ysem, tsem, osem):
    cid = lax.axis_index("c")
    sid = lax.axis_index("s")
    wid = sid * NUM_CORES + cid
    base = wid * SC_TASKS_PER_W
    lane = lax.iota(jnp.int32, LANES)

    def locate(k):
        task = base + k
        return task // NSLAB, pl.multiple_of((task % NSLAB) * SLAB, SLAB)

    def y_desc(k):
        bi, c0 = locate(k)
        return pltpu.make_async_copy(yb_hbm.at[bi, :, pl.ds(c0, SLAB)],
                                     ytab, ysem)

    def t_desc(k, s, j):
        bi, c0 = locate(k)
        h0 = pl.multiple_of(s * SB, 8)
        return pltpu.make_async_copy(
            t_hbm.at[bi, pl.ds(h0, SB), pl.ds(c0, SLAB)], tbuf.at[j],
            tsem.at[j])

    def out_desc(k, s, j):
        bi, c0 = locate(k)
        h0 = pl.multiple_of(s * SB, 8)
        return pltpu.make_async_copy(
            obuf.at[j], out_hbm.at[bi, pl.ds(h0, SB), pl.ds(c0, SLAB)],
            osem.at[j])

    def strip_compute(j):
        @plsc.parallel_loop(0, SB, unroll=2)
        def h_body(h):
            for g in range(NGRP):
                col = lane + g * LANES
                t = tbuf[j, h, pl.ds(g * LANES, LANES)]
                fi = jnp.minimum(t.astype(jnp.int32), H - 2)
                yf = plsc.load_gather(ytab, [fi, col])
                yc = plsc.load_gather(ytab, [fi + 1, col])
                frac = t - fi.astype(jnp.float32)
                obuf[j, h, pl.ds(g * LANES, LANES)] = yf + (yc - yf) * frac

    def task_body(k, carry):
        y_desc(k).start()
        t_desc(k, 0, 0).start()
        y_desc(k).wait()

        def pair_body(p, c2):
            for j in (0, 1):
                s = 2 * p + j

                @pl.when(s + 1 < NSTRIP)
                def _():
                    t_desc(k, s + 1, 1 - j).start()

                t_desc(k, s, j).wait()

                @pl.when(s >= 2)
                def _():
                    out_desc(k, s - 2, j).wait()

                strip_compute(j)
                out_desc(k, s, j).start()
            return c2

        lax.fori_loop(0, NSTRIP // 2, pair_body, 0)
        out_desc(k, NSTRIP - 2, 0).wait()
        out_desc(k, NSTRIP - 1, 1).wait()
        return carry

    lax.fori_loop(0, SC_TASKS_PER_W, task_body, 0)


def _warp_gather(y_base, t):
    mesh = plsc.VectorSubcoreMesh(core_axis_name="c", subcore_axis_name="s")
    return pl.kernel(
        _gather_body,
        out_type=jax.ShapeDtypeStruct((B, H, C), jnp.float32),
        mesh=mesh,
        scratch_types=[
            pltpu.VMEM((H, SLAB), jnp.float32),
            pltpu.VMEM((2, SB, SLAB), jnp.float32),
            pltpu.VMEM((2, SB, SLAB), jnp.float32),
            pltpu.SemaphoreType.DMA,
            pltpu.SemaphoreType.DMA((2,)),
            pltpu.SemaphoreType.DMA((2,)),
        ],
        compiler_params=pltpu.CompilerParams(use_tc_tiling_on_sc=True,
                                             needs_layout_passes=False),
    )(y_base, t)


def kernel(y_norm, y_base, future_state_hat, future_state_time,
           mu_base_fut, std_base_fut, base_time_mean, base_time_std, W, b):
    t = _compute_t(future_state_time, W, b.reshape(1, C))
    return _warp_gather(y_base, t)


# speedup vs baseline: 8.9211x; 1.0677x over previous
import functools

import jax
import jax.numpy as jnp
from jax import lax
from jax.experimental import pallas as pl
from jax.experimental.pallas import tpu as pltpu
from jax.experimental.pallas import tpu_sc as plsc

B, H, C = 64, 720, 512
EPS = 1e-4
LANES = 16
NUM_CORES = 2
NUM_SUBCORES = 16
NW = NUM_CORES * NUM_SUBCORES

NCHUNK = 4
BC = B // NCHUNK


def _tau_body(fst_ref, w_ref, b_ref, t_ref):
    x = fst_ref[0]
    raw = lax.dot_general(x, w_ref[...], (((1,), (1,)), ((), ())),
                          preferred_element_type=jnp.float32) + b_ref[...]
    sp = jnp.maximum(raw, 0.0) + jnp.log1p(jnp.exp(-jnp.abs(raw)))
    v = sp + EPS
    tau = v
    k = 1
    while k < H:
        shifted = jnp.concatenate(
            [jnp.zeros((k, C), jnp.float32), tau[:H - k]], axis=0)
        tau = tau + shifted
        k *= 2
    tau_min = tau[0:1, :]
    tau_max = tau[H - 1:H, :]
    t = (tau - tau_min) / (tau_max - tau_min + 1e-8) * (H - 1)
    t_ref[0] = jnp.clip(t, 0.0, float(H - 1))


def _compute_t_chunk(fst, w, b2, off):
    return pl.pallas_call(
        _tau_body,
        grid=(BC,),
        in_specs=[
            pl.BlockSpec((1, H, C), lambda i: (i + off, 0, 0)),
            pl.BlockSpec((C, C), lambda i: (0, 0)),
            pl.BlockSpec((1, C), lambda i: (0, 0)),
        ],
        out_specs=pl.BlockSpec((1, H, C), lambda i: (i, 0, 0)),
        out_shape=jax.ShapeDtypeStruct((BC, H, C), jnp.float32),
    )(fst, w, b2)


SLAB = 128
NSLAB = C // SLAB
SC_TASKS = BC * NSLAB
SC_TASKS_PER_W = SC_TASKS // NW
SB = 40
NSTRIP = H // SB
NGRP = SLAB // LANES


def _gather_body(off, yb_hbm, t_hbm, out_hbm, ytab, tbuf, obuf,
                 ysem, tsem, osem):
    cid = lax.axis_index("c")
    sid = lax.axis_index("s")
    wid = sid * NUM_CORES + cid
    base = wid * SC_TASKS_PER_W
    lane = lax.iota(jnp.int32, LANES)

    def locate(k):
        task = base + k
        return task // NSLAB, pl.multiple_of((task % NSLAB) * SLAB, SLAB)

    def y_desc(k):
        bi, c0 = locate(k)
        return pltpu.make_async_copy(yb_hbm.at[bi + off, :, pl.ds(c0, SLAB)],
                                     ytab, ysem)

    def t_desc(k, s, j):
        bi, c0 = locate(k)
        h0 = pl.multiple_of(s * SB, 8)
        return pltpu.make_async_copy(
            t_hbm.at[bi, pl.ds(h0, SB), pl.ds(c0, SLAB)], tbuf.at[j],
            tsem.at[j])

    def out_desc(k, s, j):
        bi, c0 = locate(k)
        h0 = pl.multiple_of(s * SB, 8)
        return pltpu.make_async_copy(
            obuf.at[j], out_hbm.at[bi, pl.ds(h0, SB), pl.ds(c0, SLAB)],
            osem.at[j])

    def strip_compute(j):
        @plsc.parallel_loop(0, SB, unroll=2)
        def h_body(h):
            for g in range(NGRP):
                col = lane + g * LANES
                t = tbuf[j, h, pl.ds(g * LANES, LANES)]
                fi = jnp.minimum(t.astype(jnp.int32), H - 2)
                yf = plsc.load_gather(ytab, [fi, col])
                yc = plsc.load_gather(ytab, [fi + 1, col])
                frac = t - fi.astype(jnp.float32)
                obuf[j, h, pl.ds(g * LANES, LANES)] = yf + (yc - yf) * frac

    def task_body(k, carry):
        y_desc(k).start()
        t_desc(k, 0, 0).start()
        y_desc(k).wait()

        def pair_body(p, c2):
            for j in (0, 1):
                s = 2 * p + j

                @pl.when(s + 1 < NSTRIP)
                def _():
                    t_desc(k, s + 1, 1 - j).start()

                t_desc(k, s, j).wait()

                @pl.when(s >= 2)
                def _():
                    out_desc(k, s - 2, j).wait()

                strip_compute(j)
                out_desc(k, s, j).start()
            return c2

        lax.fori_loop(0, NSTRIP // 2, pair_body, 0)
        out_desc(k, NSTRIP - 2, 0).wait()
        out_desc(k, NSTRIP - 1, 1).wait()
        return carry

    lax.fori_loop(0, SC_TASKS_PER_W, task_body, 0)


def _warp_gather_chunk(y_base, t, off):
    mesh = plsc.VectorSubcoreMesh(core_axis_name="c", subcore_axis_name="s")
    return pl.kernel(
        functools.partial(_gather_body, off),
        out_type=jax.ShapeDtypeStruct((BC, H, C), jnp.float32),
        mesh=mesh,
        scratch_types=[
            pltpu.VMEM((H, SLAB), jnp.float32),
            pltpu.VMEM((2, SB, SLAB), jnp.float32),
            pltpu.VMEM((2, SB, SLAB), jnp.float32),
            pltpu.SemaphoreType.DMA,
            pltpu.SemaphoreType.DMA((2,)),
            pltpu.SemaphoreType.DMA((2,)),
        ],
        compiler_params=pltpu.CompilerParams(use_tc_tiling_on_sc=True,
                                             needs_layout_passes=False),
    )(y_base, t)


def kernel(y_norm, y_base, future_state_hat, future_state_time,
           mu_base_fut, std_base_fut, base_time_mean, base_time_std, W, b):
    b2 = b.reshape(1, C)
    outs = []
    for i in range(NCHUNK):
        t = _compute_t_chunk(future_state_time, W, b2, i * BC)
        outs.append(_warp_gather_chunk(y_base, t, i * BC))
    return jnp.concatenate(outs, axis=0)


# speedup vs baseline: 9.3262x; 1.0454x over previous
import functools

import jax
import jax.numpy as jnp
from jax import lax
from jax.experimental import pallas as pl
from jax.experimental.pallas import tpu as pltpu
from jax.experimental.pallas import tpu_sc as plsc

B, H, C = 64, 720, 512
EPS = 1e-4
LANES = 16
NUM_CORES = 2
NUM_SUBCORES = 16
NW = NUM_CORES * NUM_SUBCORES

NCHUNK = 8
BC = B // NCHUNK


def _tau_body(fst_ref, w_ref, b_ref, t_ref):
    x = fst_ref[0]
    raw = lax.dot_general(x, w_ref[...], (((1,), (1,)), ((), ())),
                          preferred_element_type=jnp.float32) + b_ref[...]
    sp = jnp.maximum(raw, 0.0) + jnp.log1p(jnp.exp(-jnp.abs(raw)))
    v = sp + EPS
    tau = v
    k = 1
    while k < H:
        shifted = jnp.concatenate(
            [jnp.zeros((k, C), jnp.float32), tau[:H - k]], axis=0)
        tau = tau + shifted
        k *= 2
    tau_min = tau[0:1, :]
    tau_max = tau[H - 1:H, :]
    t = (tau - tau_min) / (tau_max - tau_min + 1e-8) * (H - 1)
    t_ref[0] = jnp.clip(t, 0.0, float(H - 1))


def _compute_t_chunk(fst, w, b2, off):
    return pl.pallas_call(
        _tau_body,
        grid=(BC,),
        in_specs=[
            pl.BlockSpec((1, H, C), lambda i: (i + off, 0, 0)),
            pl.BlockSpec((C, C), lambda i: (0, 0)),
            pl.BlockSpec((1, C), lambda i: (0, 0)),
        ],
        out_specs=pl.BlockSpec((1, H, C), lambda i: (i, 0, 0)),
        out_shape=jax.ShapeDtypeStruct((BC, H, C), jnp.float32),
    )(fst, w, b2)


SLAB = 128
NSLAB = C // SLAB
SC_TASKS = BC * NSLAB
SC_TASKS_PER_W = SC_TASKS // NW
SB = 40
NSTRIP = H // SB
NGRP = SLAB // LANES


def _gather_body(off, yb_hbm, t_hbm, out_hbm, ytab, tbuf, obuf,
                 ysem, tsem, osem):
    cid = lax.axis_index("c")
    sid = lax.axis_index("s")
    wid = sid * NUM_CORES + cid
    base = wid * SC_TASKS_PER_W
    lane = lax.iota(jnp.int32, LANES)

    def locate(k):
        task = base + k
        return task // NSLAB, pl.multiple_of((task % NSLAB) * SLAB, SLAB)

    def y_desc(k):
        bi, c0 = locate(k)
        return pltpu.make_async_copy(yb_hbm.at[bi + off, :, pl.ds(c0, SLAB)],
                                     ytab, ysem)

    def t_desc(k, s, j):
        bi, c0 = locate(k)
        h0 = pl.multiple_of(s * SB, 8)
        return pltpu.make_async_copy(
            t_hbm.at[bi, pl.ds(h0, SB), pl.ds(c0, SLAB)], tbuf.at[j],
            tsem.at[j])

    def out_desc(k, s, j):
        bi, c0 = locate(k)
        h0 = pl.multiple_of(s * SB, 8)
        return pltpu.make_async_copy(
            obuf.at[j], out_hbm.at[bi, pl.ds(h0, SB), pl.ds(c0, SLAB)],
            osem.at[j])

    def strip_compute(j):
        @plsc.parallel_loop(0, SB, unroll=2)
        def h_body(h):
            for g in range(NGRP):
                col = lane + g * LANES
                t = tbuf[j, h, pl.ds(g * LANES, LANES)]
                fi = jnp.minimum(t.astype(jnp.int32), H - 2)
                yf = plsc.load_gather(ytab, [fi, col])
                yc = plsc.load_gather(ytab, [fi + 1, col])
                frac = t - fi.astype(jnp.float32)
                obuf[j, h, pl.ds(g * LANES, LANES)] = yf + (yc - yf) * frac

    def task_body(k, carry):
        y_desc(k).start()
        t_desc(k, 0, 0).start()
        y_desc(k).wait()

        def pair_body(p, c2):
            for j in (0, 1):
                s = 2 * p + j

                @pl.when(s + 1 < NSTRIP)
                def _():
                    t_desc(k, s + 1, 1 - j).start()

                t_desc(k, s, j).wait()

                @pl.when(s >= 2)
                def _():
                    out_desc(k, s - 2, j).wait()

                strip_compute(j)
                out_desc(k, s, j).start()
            return c2

        lax.fori_loop(0, NSTRIP // 2, pair_body, 0)
        out_desc(k, NSTRIP - 2, 0).wait()
        out_desc(k, NSTRIP - 1, 1).wait()
        return carry

    lax.fori_loop(0, SC_TASKS_PER_W, task_body, 0)


def _warp_gather_chunk(y_base, t, off):
    mesh = plsc.VectorSubcoreMesh(core_axis_name="c", subcore_axis_name="s")
    return pl.kernel(
        functools.partial(_gather_body, off),
        out_type=jax.ShapeDtypeStruct((BC, H, C), jnp.float32),
        mesh=mesh,
        scratch_types=[
            pltpu.VMEM((H, SLAB), jnp.float32),
            pltpu.VMEM((2, SB, SLAB), jnp.float32),
            pltpu.VMEM((2, SB, SLAB), jnp.float32),
            pltpu.SemaphoreType.DMA,
            pltpu.SemaphoreType.DMA((2,)),
            pltpu.SemaphoreType.DMA((2,)),
        ],
        compiler_params=pltpu.CompilerParams(use_tc_tiling_on_sc=True,
                                             needs_layout_passes=False),
    )(y_base, t)


def kernel(y_norm, y_base, future_state_hat, future_state_time,
           mu_base_fut, std_base_fut, base_time_mean, base_time_std, W, b):
    b2 = b.reshape(1, C)
    outs = []
    for i in range(NCHUNK):
        t = _compute_t_chunk(future_state_time, W, b2, i * BC)
        outs.append(_warp_gather_chunk(y_base, t, i * BC))
    return jnp.concatenate(outs, axis=0)
